# baseline (speedup 1.0000x reference)
import functools

import jax
import jax.numpy as jnp
import numpy as np
from jax import lax
from jax.experimental import pallas as pl
from jax.experimental.pallas import tpu as pltpu
from jax.experimental.pallas import tpu_sc as plsc

_TOKENS = 32768
_D = 512
_KEEP_PROB = 0.9

_BLK = 4096


def _rotl(x, d):
    return ((x << np.uint32(d)) | (x >> np.uint32(32 - d))).astype(np.uint32)


def _threefry2x32(k1, k2, x0, x1):
    rot = [np.uint32(r) for r in (13, 15, 26, 6, 17, 29, 16, 24)]
    r0, r1 = rot[:4], rot[4:]
    ks0, ks1 = np.uint32(k1), np.uint32(k2)
    ks2 = ks0 ^ ks1 ^ np.uint32(0x1BD11BDA)
    x0 = (x0 + ks0).astype(np.uint32)
    x1 = (x1 + ks1).astype(np.uint32)

    def rounds(x0, x1, rots):
        for r in rots:
            x0 = (x0 + x1).astype(np.uint32)
            x1 = _rotl(x1, r) ^ x0
        return x0, x1

    x0, x1 = rounds(x0, x1, r0)
    x0 = (x0 + ks1).astype(np.uint32)
    x1 = (x1 + ks2 + np.uint32(1)).astype(np.uint32)
    x0, x1 = rounds(x0, x1, r1)
    x0 = (x0 + ks2).astype(np.uint32)
    x1 = (x1 + ks0 + np.uint32(2)).astype(np.uint32)
    x0, x1 = rounds(x0, x1, r0)
    x0 = (x0 + ks0).astype(np.uint32)
    x1 = (x1 + ks1 + np.uint32(3)).astype(np.uint32)
    x0, x1 = rounds(x0, x1, r1)
    x0 = (x0 + ks1).astype(np.uint32)
    x1 = (x1 + ks2 + np.uint32(4)).astype(np.uint32)
    x0, x1 = rounds(x0, x1, r0)
    x0 = (x0 + ks2).astype(np.uint32)
    x1 = (x1 + ks0 + np.uint32(5)).astype(np.uint32)
    return x0, x1


def _dropout_mask():
    n, seed = _TOKENS, 42
    if jax.config.jax_threefry_partitionable:
        y0, y1 = _threefry2x32(0, seed, np.zeros(n, np.uint32),
                               np.arange(n, dtype=np.uint32))
        bits = y0 ^ y1
    else:
        cnt = np.arange(n, dtype=np.uint32)
        y0, y1 = _threefry2x32(0, seed, cnt[: n // 2], cnt[n // 2:])
        bits = np.concatenate([y0, y1])
    fb = (bits >> np.uint32(9)) | np.uint32(0x3F800000)
    u = fb.view(np.float32) - np.float32(1.0)
    return u < np.float32(_KEEP_PROB)


def _mask_body(x_ref, m_ref, o_ref):
    x = x_ref[...].reshape(_BLK // 128, 128, _D)
    m = m_ref[...].reshape(_BLK // 128, 128, 1)
    o_ref[...] = (x * m).reshape(_BLK, _D)


def _kernel_tc(flat):
    mask = jnp.asarray(
        _dropout_mask().astype(np.float32).reshape(_TOKENS // 128, 128))
    grid = _TOKENS // _BLK
    return pl.pallas_call(
        _mask_body,
        grid=(grid,),
        in_specs=[
            pl.BlockSpec((_BLK, _D), lambda i: (i, 0)),
            pl.BlockSpec((_BLK // 128, 128), lambda i: (i, 0)),
        ],
        out_specs=pl.BlockSpec((_BLK, _D), lambda i: (i, 0)),
        out_shape=jax.ShapeDtypeStruct((_TOKENS, _D), jnp.float32),
    )(flat, mask)


_NC, _NS = 2, 16
_NW = _NC * _NS
_CK = 32
_CD = 16
_RING = 4


def _pad_split(idx, chunk):
    per_w = -(-len(idx) // (_NW * chunk)) * chunk
    total = per_w * _NW
    pad = np.resize(idx[-1:], total - len(idx)) if total > len(idx) else idx[:0]
    return np.concatenate([idx, pad]).astype(np.int32).reshape(_NW, per_w // chunk, chunk)


def _kernel_sc(flat):
    mask = _dropout_mask()
    kept3 = _pad_split(np.flatnonzero(mask), _CK)
    drop3 = _pad_split(np.flatnonzero(~mask), _CD)
    nch, ndch = kept3.shape[1], drop3.shape[1]

    mesh = plsc.VectorSubcoreMesh(core_axis_name="c", subcore_axis_name="s")

    @functools.partial(
        pl.kernel,
        out_type=jax.ShapeDtypeStruct((_TOKENS, _D), jnp.float32),
        mesh=mesh,
        scratch_types=(
            [pltpu.VMEM((nch, _CK), jnp.int32),
             pltpu.VMEM((ndch, _CD), jnp.int32),
             pltpu.VMEM((_CD, _D), jnp.float32)]
            + [pltpu.VMEM((_CK, _D), jnp.float32) for _ in range(_RING)]
            + [pltpu.SemaphoreType.DMA for _ in range(2 * _RING + 1)]
        ),
    )
    def body(flat_hbm, kidx_hbm, didx_hbm, out_hbm, kidx_v, didx_v, zbuf,
             *bufs_and_sems):
        bufs = bufs_and_sems[:_RING]
        gsem = bufs_and_sems[_RING:2 * _RING]
        ssem = bufs_and_sems[2 * _RING:3 * _RING]
        zsem = bufs_and_sems[3 * _RING]
        wid = lax.axis_index("s") * _NC + lax.axis_index("c")

        pltpu.sync_copy(kidx_hbm.at[wid], kidx_v)
        pltpu.sync_copy(didx_hbm.at[wid], didx_v)

        zero = jnp.zeros((16,), jnp.float32)
        for r in range(_CD):
            for k in range(_D // 16):
                zbuf[r, pl.ds(k * 16, 16)] = zero
        for j in range(ndch):
            pltpu.async_copy(zbuf, out_hbm.at[didx_v.at[j]], zsem).wait()

        g = [None] * nch
        s = [None] * nch
        waited = [False] * nch
        for j in range(nch + 1):
            if j < nch:
                b = j % _RING
                if j >= _RING:
                    s[j - _RING].wait()
                    waited[j - _RING] = True
                g[j] = pltpu.async_copy(flat_hbm.at[kidx_v.at[j]], bufs[b], gsem[b])
            if j >= 1:
                i = j - 1
                g[i].wait()
                s[i] = pltpu.async_copy(bufs[i % _RING], out_hbm.at[kidx_v.at[i]],
                                        ssem[i % _RING])
        for i in range(nch):
            if not waited[i]:
                s[i].wait()

    return body(flat, jnp.asarray(kept3), jnp.asarray(drop3))


def kernel(flat, row_starts):
    del row_starts
    return _kernel_sc(flat)
